# baseline (speedup 1.0000x reference)
import functools

import jax
import jax.numpy as jnp
from jax import lax
from jax.experimental import pallas as pl
from jax.experimental.pallas import tpu as pltpu
from jax.experimental.pallas import tpu_sc as plsc

HIDDEN_DIM = 64
GAMMA = 12.0
BATCH = 16384

_INFO = plsc.get_sparse_core_info()
_NC = _INFO.num_cores
_NS = _INFO.num_subcores
_NW = _NC * _NS
_BPW = BATCH // _NW
_CHUNK = 128
_NCHUNK = _BPW // _CHUNK
_GROUPS = _BPW // 16
_GPC = _CHUNK // 16


def _make_kernel():
    mesh = plsc.VectorSubcoreMesh(core_axis_name="c", subcore_axis_name="s")

    @functools.partial(
        pl.kernel,
        mesh=mesh,
        out_type=jax.ShapeDtypeStruct((BATCH,), jnp.float32),
        scratch_types=[
            pltpu.VMEM((_BPW, 3), jnp.int32),
            pltpu.VMEM((_NCHUNK, _CHUNK), jnp.int32),
            pltpu.VMEM((_NCHUNK, _CHUNK), jnp.int32),
            pltpu.VMEM((_NCHUNK, _CHUNK), jnp.int32),
            pltpu.VMEM((_BPW, HIDDEN_DIM), jnp.float32),
            pltpu.VMEM((_BPW, HIDDEN_DIM), jnp.float32),
            pltpu.VMEM((_BPW, HIDDEN_DIM), jnp.float32),
            pltpu.VMEM((_BPW,), jnp.float32),
            pltpu.SemaphoreType.DMA,
        ],
        compiler_params=pltpu.CompilerParams(
            needs_layout_passes=False, use_tc_tiling_on_sc=False),
    )
    def kge_score(sample_hbm, ent_hbm, rel_hbm, out_hbm,
                  trip_v, idx_h, idx_r, idx_t,
                  rows_h, rows_r, rows_t, out_v, sem):
        wid = lax.axis_index("s") * _NC + lax.axis_index("c")
        base = wid * _BPW

        pltpu.sync_copy(sample_hbm.at[pl.ds(base, _BPW), :], trip_v)

        lane = lax.iota(jnp.int32, 16)
        col0 = jnp.zeros((16,), jnp.int32)
        col1 = jnp.full((16,), 1, jnp.int32)
        col2 = jnp.full((16,), 2, jnp.int32)

        copies = []
        for j in range(_NCHUNK):
            for gg in range(_GPC):
                g = j * _GPC + gg
                rid = g * 16 + lane
                c = pl.ds(gg * 16, 16)
                idx_h[j, c] = plsc.load_gather(trip_v, [rid, col0])
                idx_r[j, c] = plsc.load_gather(trip_v, [rid, col1])
                idx_t[j, c] = plsc.load_gather(trip_v, [rid, col2])
            dst = pl.ds(j * _CHUNK, _CHUNK)
            copies.append(
                pltpu.async_copy(ent_hbm.at[idx_h.at[j]], rows_h.at[dst], sem))
            copies.append(
                pltpu.async_copy(rel_hbm.at[idx_r.at[j]], rows_r.at[dst], sem))
            copies.append(
                pltpu.async_copy(ent_hbm.at[idx_t.at[j]], rows_t.at[dst], sem))
        for c in copies:
            c.wait()

        def group_body(g, carry):
            rids = g * 16 + lane
            acc = jnp.zeros((16,), jnp.float32)
            for d in range(HIDDEN_DIM):
                cold = jnp.full((16,), d, jnp.int32)
                hv = plsc.load_gather(rows_h, [rids, cold])
                rv = plsc.load_gather(rows_r, [rids, cold])
                tv = plsc.load_gather(rows_t, [rids, cold])
                acc = acc + jnp.abs(hv + rv - tv)
            out_v[pl.ds(g * 16, 16)] = GAMMA - acc
            return carry

        lax.fori_loop(0, _GROUPS, group_body, 0)

        pltpu.sync_copy(out_v, out_hbm.at[pl.ds(base, _BPW)])

    return kge_score


_KERNEL = _make_kernel()


def kernel(sample, entity_embedding, relation_embedding):
    scores = _KERNEL(sample, entity_embedding, relation_embedding)
    return scores.reshape(BATCH, 1)


# speedup vs baseline: 1.0634x; 1.0634x over previous
import functools

import jax
import jax.numpy as jnp
from jax import lax
from jax.experimental import pallas as pl
from jax.experimental.pallas import tpu as pltpu
from jax.experimental.pallas import tpu_sc as plsc

HIDDEN_DIM = 64
PADDED_DIM = 128
GAMMA = 12.0
BATCH = 16384

_INFO = plsc.get_sparse_core_info()
_NC = _INFO.num_cores
_NS = _INFO.num_subcores
_NW = _NC * _NS
_BPW = BATCH // _NW
_HALF = _BPW // 2
_CHUNK = 128
_NCHUNK = _BPW // _CHUNK
_GROUPS = _HALF // 16


def _make_kernel():
    mesh = plsc.VectorSubcoreMesh(core_axis_name="c", subcore_axis_name="s")

    @functools.partial(
        pl.kernel,
        mesh=mesh,
        out_type=jax.ShapeDtypeStruct((BATCH,), jnp.float32),
        scratch_types=[
            pltpu.VMEM((_NCHUNK, 1, _CHUNK), jnp.int32),
            pltpu.VMEM((_NCHUNK, 1, _CHUNK), jnp.int32),
            pltpu.VMEM((_NCHUNK, 1, _CHUNK), jnp.int32),
            pltpu.VMEM((_HALF, PADDED_DIM), jnp.float32),
            pltpu.VMEM((_HALF, PADDED_DIM), jnp.float32),
            pltpu.VMEM((_HALF, PADDED_DIM), jnp.float32),
            pltpu.VMEM((_BPW,), jnp.float32),
            pltpu.SemaphoreType.DMA,
        ],
        compiler_params=pltpu.CompilerParams(
            needs_layout_passes=False, use_tc_tiling_on_sc=True),
    )
    def kge_score(h_hbm, r_hbm, t_hbm, ent_hbm, rel_hbm, out_hbm,
                  idx_h, idx_r, idx_t, rows_h, rows_r, rows_t, out_v, sem):
        wid = lax.axis_index("s") * _NC + lax.axis_index("c")
        base = wid * _BPW

        stage = []
        for j in range(_NCHUNK):
            src = pl.ds(base + j * _CHUNK, _CHUNK)
            stage.append(pltpu.async_copy(h_hbm.at[src], idx_h.at[j, 0], sem))
            stage.append(pltpu.async_copy(r_hbm.at[src], idx_r.at[j, 0], sem))
            stage.append(pltpu.async_copy(t_hbm.at[src], idx_t.at[j, 0], sem))
        for c in stage:
            c.wait()

        lane = lax.iota(jnp.int32, 16)

        for half in range(2):
            copies = []
            for j in range(2):
                chunk = half * 2 + j
                dst = pl.ds(j * _CHUNK, _CHUNK)
                copies.append(pltpu.async_copy(
                    ent_hbm.at[idx_h.at[chunk, 0]], rows_h.at[dst], sem))
                copies.append(pltpu.async_copy(
                    rel_hbm.at[idx_r.at[chunk, 0]], rows_r.at[dst], sem))
                copies.append(pltpu.async_copy(
                    ent_hbm.at[idx_t.at[chunk, 0]], rows_t.at[dst], sem))
            for c in copies:
                c.wait()

            def group_body(g, carry):
                rids = g * 16 + lane
                acc = jnp.zeros((16,), jnp.float32)
                for d in range(HIDDEN_DIM):
                    cold = jnp.full((16,), d, jnp.int32)
                    hv = plsc.load_gather(rows_h, [rids, cold])
                    rv = plsc.load_gather(rows_r, [rids, cold])
                    tv = plsc.load_gather(rows_t, [rids, cold])
                    acc = acc + jnp.abs(hv + rv - tv)
                out_v[pl.ds(half * _HALF + g * 16, 16)] = GAMMA - acc
                return carry

            lax.fori_loop(0, _GROUPS, group_body, 0)

        pltpu.sync_copy(out_v, out_hbm.at[pl.ds(base, _BPW)])

    return kge_score


_KERNEL = _make_kernel()


def kernel(sample, entity_embedding, relation_embedding):
    ent_p = jnp.pad(entity_embedding, ((0, 0), (0, PADDED_DIM - HIDDEN_DIM)))
    rel_p = jnp.pad(relation_embedding, ((0, 0), (0, PADDED_DIM - HIDDEN_DIM)))
    scores = _KERNEL(sample[:, 0], sample[:, 1], sample[:, 2], ent_p, rel_p)
    return scores.reshape(BATCH, 1)
